# Initial kernel scaffold; baseline (speedup 1.0000x reference)
import jax
import jax.numpy as jnp
from jax.experimental import pallas as pl


def kernel(x, edge_index, edge_attr, We1, be1, W1a, b1a, W1b, b1b, g1, bt1, pa1, We2, be2, W2a, b2a, W2b, b2b, g2, bt2, pa2, We3, be3, W3a, b3a, W3b, b3b, g3, bt3, pa3, Wo, bo):
    raise NotImplementedError("write your pallas kernel here")


# speedup vs baseline: 1.2913x; 1.2913x over previous
import functools

import jax
import jax.numpy as jnp
from jax import lax
from jax.experimental import pallas as pl
from jax.experimental.pallas import tpu as pltpu
from jax.experimental.pallas import tpu_sc as plsc

N = 10000
E = 320000
D = 128
ED = 16

NC = 2
NS = 16
NW = NC * NS
N_PAD = 10240
RPW = N_PAD // NW
C = 80
CB = 96
VPR = D // 16


def _sc_body(x_hbm, e_hbm, src_hbm, dst_hbm, bounds_hbm, out_hbm,
             bounds_v, src_idx, dst_idx, xrows, erows, acc, sem):
    cid = lax.axis_index("c")
    sid = lax.axis_index("s")
    wid = sid * NC + cid
    base_row = wid * RPW

    pltpu.sync_copy(bounds_hbm, bounds_v)
    bv = bounds_v[wid, pl.ds(0, 16)]
    b_lo = bv[0]
    b_hi = bv[1]

    zero = jnp.zeros((16,), jnp.float32)

    def zrow(r, _):
        for v in range(VPR):
            acc[r, pl.ds(v * 16, 16)] = zero
        return 0

    lax.fori_loop(0, RPW, zrow, 0)

    nch = (b_hi - b_lo + C - 1) // C

    def chunk(k, _):
        logical = b_lo + k * C
        base = jnp.minimum((logical // 8) * 8, E - CB)
        r_lo = logical - base
        r_hi = jnp.minimum(logical + C, b_hi) - base
        pltpu.sync_copy(src_hbm.at[pl.ds(base, CB)], src_idx)
        pltpu.sync_copy(dst_hbm.at[pl.ds(base, CB)], dst_idx)
        gat = pltpu.async_copy(x_hbm.at[src_idx], xrows, sem)
        pltpu.sync_copy(e_hbm.at[pl.ds(base, CB)], erows)
        gat.wait()

        def group(g, _):
            gbase = g * 16
            dv = dst_idx[pl.ds(gbase, 16)] - base_row
            for l in range(16):
                r = gbase + l

                @pl.when((r >= r_lo) & (r < r_hi))
                def _():
                    d = dv[l]
                    for v in range(VPR):
                        s = pl.ds(v * 16, 16)
                        plsc.addupdate(
                            acc.at[d, s],
                            jnp.maximum(xrows[r, s] + erows[r, s], 0.0))

            return 0

        lax.fori_loop(0, CB // 16, group, 0)
        return 0

    lax.fori_loop(0, nch, chunk, 0)
    pltpu.sync_copy(acc, out_hbm.at[pl.ds(base_row, RPW)])


_sc_aggregate = functools.partial(
    pl.kernel,
    mesh=plsc.VectorSubcoreMesh(core_axis_name="c", subcore_axis_name="s"),
    out_type=jax.ShapeDtypeStruct((N_PAD, D), jnp.float32),
    scratch_types=[
        pltpu.VMEM((NW, 16), jnp.int32),
        pltpu.VMEM((CB,), jnp.int32),
        pltpu.VMEM((CB,), jnp.int32),
        pltpu.VMEM((CB, D), jnp.float32),
        pltpu.VMEM((CB, D), jnp.float32),
        pltpu.VMEM((RPW, D), jnp.float32),
        pltpu.SemaphoreType.DMA,
    ],
)(_sc_body)


_BE = 2000


def _edge_mlp_body(ea_ref, we_ref, be_ref, out_ref):
    out_ref[...] = lax.dot_general(
        ea_ref[...], we_ref[...], (((1,), (1,)), ((), ())),
        preferred_element_type=jnp.float32) + be_ref[...]


def _edge_mlp(edge_attr, We, be):
    return pl.pallas_call(
        _edge_mlp_body,
        grid=(E // _BE,),
        in_specs=[
            pl.BlockSpec((_BE, ED), lambda i: (i, 0)),
            pl.BlockSpec((D, ED), lambda i: (0, 0)),
            pl.BlockSpec((1, D), lambda i: (0, 0)),
        ],
        out_specs=pl.BlockSpec((_BE, D), lambda i: (i, 0)),
        out_shape=jax.ShapeDtypeStruct((E, D), jnp.float32),
    )(edge_attr, We, be.reshape(1, D))


def _node_core(x, agg, wa, ba, wb, bb, g, bt, pa):
    h = x + agg
    h = jnp.maximum(
        lax.dot_general(h, wa, (((1,), (1,)), ((), ())),
                        preferred_element_type=jnp.float32) + ba, 0.0)
    h = lax.dot_general(h, wb, (((1,), (1,)), ((), ())),
                        preferred_element_type=jnp.float32) + bb
    mu = jnp.mean(h, axis=0, keepdims=True)
    var = jnp.mean((h - mu) * (h - mu), axis=0, keepdims=True)
    h = (h - mu) * lax.rsqrt(var + 1e-5) * g + bt
    return jnp.where(h > 0, h, pa * h)


def _node_mid_body(x_ref, a_ref, wa_ref, ba_ref, wb_ref, bb_ref,
                   g_ref, bt_ref, pa_ref, out_ref):
    out_ref[...] = _node_core(
        x_ref[...], a_ref[:N, :], wa_ref[...], ba_ref[...],
        wb_ref[...], bb_ref[...], g_ref[...], bt_ref[...], pa_ref[...])


def _node_last_body(x_ref, a_ref, wa_ref, ba_ref, wb_ref, bb_ref,
                    g_ref, bt_ref, pa_ref, wo_ref, bo_ref, out_ref):
    h = _node_core(
        x_ref[...], a_ref[:N, :], wa_ref[...], ba_ref[...],
        wb_ref[...], bb_ref[...], g_ref[...], bt_ref[...], pa_ref[...])
    out_ref[...] = lax.dot_general(
        h, wo_ref[...], (((1,), (1,)), ((), ())),
        preferred_element_type=jnp.float32) + bo_ref[...]


def _node_update(x, agg, wa, ba, wb, bb, g, bt, pa, wo=None, bo=None):
    row = lambda v: v.reshape(1, D)
    args = [x, agg, wa, row(ba), wb, row(bb), row(g), row(bt), row(pa)]
    body = _node_mid_body
    if wo is not None:
        args += [wo, row(bo)]
        body = _node_last_body
    return pl.pallas_call(
        body,
        out_shape=jax.ShapeDtypeStruct((N, D), jnp.float32),
    )(*args)


def kernel(x, edge_index, edge_attr,
           We1, be1, W1a, b1a, W1b, b1b, g1, bt1, pa1,
           We2, be2, W2a, b2a, W2b, b2b, g2, bt2, pa2,
           We3, be3, W3a, b3a, W3b, b3b, g3, bt3, pa3,
           Wo, bo):
    perm = jnp.argsort(edge_index[1])
    src_s = jnp.take(edge_index[0], perm)
    dst_s = jnp.take(edge_index[1], perm)
    ea_s = jnp.take(edge_attr, perm, axis=0)
    bins = jnp.arange(0, N_PAD + RPW, RPW, dtype=jnp.int32)
    b = jnp.searchsorted(dst_s, bins).astype(jnp.int32)
    bounds = jnp.zeros((NW, 16), jnp.int32).at[:, 0].set(b[:-1]).at[:, 1].set(b[1:])

    e1 = _edge_mlp(ea_s, We1, be1)
    agg = _sc_aggregate(x, e1, src_s, dst_s, bounds)
    h = _node_update(x, agg, W1a, b1a, W1b, b1b, g1, bt1, pa1)

    e2 = _edge_mlp(ea_s, We2, be2)
    agg = _sc_aggregate(h, e2, src_s, dst_s, bounds)
    h = _node_update(h, agg, W2a, b2a, W2b, b2b, g2, bt2, pa2)

    e3 = _edge_mlp(ea_s, We3, be3)
    agg = _sc_aggregate(h, e3, src_s, dst_s, bounds)
    return _node_update(h, agg, W3a, b3a, W3b, b3b, g3, bt3, pa3, Wo, bo)


# speedup vs baseline: 1.8059x; 1.3985x over previous
import functools

import jax
import jax.numpy as jnp
from jax import lax
from jax.experimental import pallas as pl
from jax.experimental.pallas import tpu as pltpu
from jax.experimental.pallas import tpu_sc as plsc

N = 10000
E = 320000
D = 128
ED = 16

NC = 2
NS = 16
NW = NC * NS
N_PAD = 10240
RPW = N_PAD // NW
C = 80
CB = 96
VPR = D // 16


def _sc_body(x_hbm, e_hbm, src_hbm, dst_hbm, prm_hbm, bounds_hbm, out_hbm,
             bounds_v,
             src0, prm0, dst0, x0, e0,
             src1, prm1, dst1, x1, e1,
             acc, semi0, semi1, semd0, semd1):
    cid = lax.axis_index("c")
    sid = lax.axis_index("s")
    wid = sid * NC + cid
    base_row = wid * RPW

    srcb = (src0, src1)
    prmb = (prm0, prm1)
    dstb = (dst0, dst1)
    xb = (x0, x1)
    eb = (e0, e1)
    semi = (semi0, semi1)
    semd = (semd0, semd1)

    pltpu.sync_copy(bounds_hbm, bounds_v)
    bv = bounds_v[wid, pl.ds(0, 16)]
    b_lo = bv[0]
    b_hi = bv[1]

    zero = jnp.zeros((16,), jnp.float32)

    def zrow(r, _):
        for v in range(VPR):
            acc[r, pl.ds(v * 16, 16)] = zero
        return 0

    lax.fori_loop(0, RPW, zrow, 0)

    nch = (b_hi - b_lo + C - 1) // C

    def chunk_base(k):
        logical = b_lo + k * C
        return jnp.minimum((logical // 8) * 8, E - CB)

    def issue_idx(k, b):
        base = chunk_base(k)
        pltpu.async_copy(src_hbm.at[pl.ds(base, CB)], srcb[b], semi[b])
        pltpu.async_copy(prm_hbm.at[pl.ds(base, CB)], prmb[b], semi[b])
        pltpu.async_copy(dst_hbm.at[pl.ds(base, CB)], dstb[b], semi[b])

    def wait_idx(k, b):
        base = chunk_base(k)
        pltpu.make_async_copy(src_hbm.at[pl.ds(base, CB)], srcb[b], semi[b]).wait()
        pltpu.make_async_copy(prm_hbm.at[pl.ds(base, CB)], prmb[b], semi[b]).wait()
        pltpu.make_async_copy(dst_hbm.at[pl.ds(base, CB)], dstb[b], semi[b]).wait()

    def issue_data(k, b):
        pltpu.async_copy(x_hbm.at[srcb[b]], xb[b], semd[b])
        pltpu.async_copy(e_hbm.at[prmb[b]], eb[b], semd[b])

    def wait_data(k, b):
        pltpu.make_async_copy(x_hbm.at[srcb[b]], xb[b], semd[b]).wait()
        pltpu.make_async_copy(e_hbm.at[prmb[b]], eb[b], semd[b]).wait()

    def compute(k, b):
        logical = b_lo + k * C
        base = chunk_base(k)
        r_lo = logical - base
        r_hi = jnp.minimum(logical + C, b_hi) - base
        xrows = xb[b]
        erows = eb[b]
        dst_idx = dstb[b]

        def group(g, _):
            gbase = g * 16
            dv = dst_idx[pl.ds(gbase, 16)] - base_row
            for l in range(16):
                r = gbase + l

                @pl.when((r >= r_lo) & (r < r_hi))
                def _():
                    d = dv[l]
                    for v in range(VPR):
                        s = pl.ds(v * 16, 16)
                        plsc.addupdate(
                            acc.at[d, s],
                            jnp.maximum(xrows[r, s] + erows[r, s], 0.0))

            return 0

        lax.fori_loop(0, CB // 16, group, 0)

    issue_idx(0, 0)
    wait_idx(0, 0)
    issue_data(0, 0)
    issue_idx(1, 1)

    npairs = (nch + 1) // 2

    def pair(j, _):
        for b in range(2):
            k = 2 * j + b
            o = 1 - b
            wait_idx(k + 1, o)
            issue_data(k + 1, o)
            wait_data(k, b)
            compute(k, b)
            issue_idx(k + 2, b)
        return 0

    lax.fori_loop(0, npairs, pair, 0)

    kk = 2 * npairs
    wait_data(kk, 0)
    wait_idx(kk + 1, 1)

    pltpu.sync_copy(acc, out_hbm.at[pl.ds(base_row, RPW)])


_sc_aggregate = functools.partial(
    pl.kernel,
    mesh=plsc.VectorSubcoreMesh(core_axis_name="c", subcore_axis_name="s"),
    out_type=jax.ShapeDtypeStruct((N_PAD, D), jnp.float32),
    scratch_types=[
        pltpu.VMEM((NW, 16), jnp.int32),
        pltpu.VMEM((CB,), jnp.int32),
        pltpu.VMEM((CB,), jnp.int32),
        pltpu.VMEM((CB,), jnp.int32),
        pltpu.VMEM((CB, D), jnp.float32),
        pltpu.VMEM((CB, D), jnp.float32),
        pltpu.VMEM((CB,), jnp.int32),
        pltpu.VMEM((CB,), jnp.int32),
        pltpu.VMEM((CB,), jnp.int32),
        pltpu.VMEM((CB, D), jnp.float32),
        pltpu.VMEM((CB, D), jnp.float32),
        pltpu.VMEM((RPW, D), jnp.float32),
        pltpu.SemaphoreType.DMA,
        pltpu.SemaphoreType.DMA,
        pltpu.SemaphoreType.DMA,
        pltpu.SemaphoreType.DMA,
    ],
)(_sc_body)


_BE = 2000


def _edge_mlp_body(ea_ref, we_ref, be_ref, out_ref):
    out_ref[...] = lax.dot_general(
        ea_ref[...], we_ref[...], (((1,), (1,)), ((), ())),
        preferred_element_type=jnp.float32) + be_ref[...]


def _edge_mlp(edge_attr, We, be):
    return pl.pallas_call(
        _edge_mlp_body,
        grid=(E // _BE,),
        in_specs=[
            pl.BlockSpec((_BE, ED), lambda i: (i, 0)),
            pl.BlockSpec((D, ED), lambda i: (0, 0)),
            pl.BlockSpec((1, D), lambda i: (0, 0)),
        ],
        out_specs=pl.BlockSpec((_BE, D), lambda i: (i, 0)),
        out_shape=jax.ShapeDtypeStruct((E, D), jnp.float32),
    )(edge_attr, We, be.reshape(1, D))


def _node_core(x, agg, wa, ba, wb, bb, g, bt, pa):
    h = x + agg
    h = jnp.maximum(
        lax.dot_general(h, wa, (((1,), (1,)), ((), ())),
                        preferred_element_type=jnp.float32) + ba, 0.0)
    h = lax.dot_general(h, wb, (((1,), (1,)), ((), ())),
                        preferred_element_type=jnp.float32) + bb
    mu = jnp.mean(h, axis=0, keepdims=True)
    var = jnp.mean((h - mu) * (h - mu), axis=0, keepdims=True)
    h = (h - mu) * lax.rsqrt(var + 1e-5) * g + bt
    return jnp.where(h > 0, h, pa * h)


def _node_mid_body(x_ref, a_ref, wa_ref, ba_ref, wb_ref, bb_ref,
                   g_ref, bt_ref, pa_ref, out_ref):
    out_ref[...] = _node_core(
        x_ref[...], a_ref[:N, :], wa_ref[...], ba_ref[...],
        wb_ref[...], bb_ref[...], g_ref[...], bt_ref[...], pa_ref[...])


def _node_last_body(x_ref, a_ref, wa_ref, ba_ref, wb_ref, bb_ref,
                    g_ref, bt_ref, pa_ref, wo_ref, bo_ref, out_ref):
    h = _node_core(
        x_ref[...], a_ref[:N, :], wa_ref[...], ba_ref[...],
        wb_ref[...], bb_ref[...], g_ref[...], bt_ref[...], pa_ref[...])
    out_ref[...] = lax.dot_general(
        h, wo_ref[...], (((1,), (1,)), ((), ())),
        preferred_element_type=jnp.float32) + bo_ref[...]


def _node_update(x, agg, wa, ba, wb, bb, g, bt, pa, wo=None, bo=None):
    row = lambda v: v.reshape(1, D)
    args = [x, agg, wa, row(ba), wb, row(bb), row(g), row(bt), row(pa)]
    body = _node_mid_body
    if wo is not None:
        args += [wo, row(bo)]
        body = _node_last_body
    return pl.pallas_call(
        body,
        out_shape=jax.ShapeDtypeStruct((N, D), jnp.float32),
    )(*args)


def kernel(x, edge_index, edge_attr,
           We1, be1, W1a, b1a, W1b, b1b, g1, bt1, pa1,
           We2, be2, W2a, b2a, W2b, b2b, g2, bt2, pa2,
           We3, be3, W3a, b3a, W3b, b3b, g3, bt3, pa3,
           Wo, bo):
    perm = jnp.argsort(edge_index[1]).astype(jnp.int32)
    src_s = jnp.take(edge_index[0], perm)
    dst_s = jnp.take(edge_index[1], perm)
    bins = jnp.arange(0, N_PAD + RPW, RPW, dtype=jnp.int32)
    b = jnp.searchsorted(dst_s, bins).astype(jnp.int32)
    bounds = jnp.zeros((NW, 16), jnp.int32).at[:, 0].set(b[:-1]).at[:, 1].set(b[1:])

    e1 = _edge_mlp(edge_attr, We1, be1)
    agg = _sc_aggregate(x, e1, src_s, dst_s, perm, bounds)
    h = _node_update(x, agg, W1a, b1a, W1b, b1b, g1, bt1, pa1)

    e2 = _edge_mlp(edge_attr, We2, be2)
    agg = _sc_aggregate(h, e2, src_s, dst_s, perm, bounds)
    h = _node_update(h, agg, W2a, b2a, W2b, b2b, g2, bt2, pa2)

    e3 = _edge_mlp(edge_attr, We3, be3)
    agg = _sc_aggregate(h, e3, src_s, dst_s, perm, bounds)
    return _node_update(h, agg, W3a, b3a, W3b, b3b, g3, bt3, pa3, Wo, bo)


# speedup vs baseline: 1.8651x; 1.0328x over previous
import functools

import jax
import jax.numpy as jnp
from jax import lax
from jax.experimental import pallas as pl
from jax.experimental.pallas import tpu as pltpu
from jax.experimental.pallas import tpu_sc as plsc

N = 10000
E = 320000
D = 128
ED = 16

NC = 2
NS = 16
NW = NC * NS
N_PAD = 10240
RPW = N_PAD // NW
C = 120
CB = 128
VPR = D // 16


def _sc_body(x_hbm, e_hbm, src_hbm, dst_hbm, prm_hbm, bounds_hbm, out_hbm,
             bounds_v,
             src0, prm0, dst0, x0, e0,
             src1, prm1, dst1, x1, e1,
             acc, semi0, semi1, semd0, semd1):
    cid = lax.axis_index("c")
    sid = lax.axis_index("s")
    wid = sid * NC + cid
    base_row = wid * RPW

    srcb = (src0, src1)
    prmb = (prm0, prm1)
    dstb = (dst0, dst1)
    xb = (x0, x1)
    eb = (e0, e1)
    semi = (semi0, semi1)
    semd = (semd0, semd1)

    pltpu.sync_copy(bounds_hbm, bounds_v)
    bv = bounds_v[wid, pl.ds(0, 16)]
    b_lo = bv[0]
    b_hi = bv[1]

    zero = jnp.zeros((16,), jnp.float32)

    def zrow(r, _):
        for v in range(VPR):
            acc[r, pl.ds(v * 16, 16)] = zero
        return 0

    lax.fori_loop(0, RPW, zrow, 0)

    nch = (b_hi - b_lo + C - 1) // C

    def chunk_base(k):
        logical = b_lo + k * C
        return jnp.minimum((logical // 8) * 8, E - CB)

    def issue_idx(k, b):
        base = chunk_base(k)
        pltpu.async_copy(src_hbm.at[pl.ds(base, CB)], srcb[b], semi[b])
        pltpu.async_copy(prm_hbm.at[pl.ds(base, CB)], prmb[b], semi[b])
        pltpu.async_copy(dst_hbm.at[pl.ds(base, CB)], dstb[b], semi[b])

    def wait_idx(k, b):
        base = chunk_base(k)
        pltpu.make_async_copy(src_hbm.at[pl.ds(base, CB)], srcb[b], semi[b]).wait()
        pltpu.make_async_copy(prm_hbm.at[pl.ds(base, CB)], prmb[b], semi[b]).wait()
        pltpu.make_async_copy(dst_hbm.at[pl.ds(base, CB)], dstb[b], semi[b]).wait()

    def issue_data(k, b):
        pltpu.async_copy(x_hbm.at[srcb[b]], xb[b], semd[b])
        pltpu.async_copy(e_hbm.at[prmb[b]], eb[b], semd[b])

    def wait_data(k, b):
        pltpu.make_async_copy(x_hbm.at[srcb[b]], xb[b], semd[b]).wait()
        pltpu.make_async_copy(e_hbm.at[prmb[b]], eb[b], semd[b]).wait()

    def compute(k, b):
        logical = b_lo + k * C
        base = chunk_base(k)
        r_lo = logical - base
        r_hi = jnp.minimum(logical + C, b_hi) - base
        xrows = xb[b]
        erows = eb[b]
        dst_idx = dstb[b]

        def group(g, _):
            gbase = g * 16
            dv = dst_idx[pl.ds(gbase, 16)] - base_row

            def edge(l, guarded):
                r = gbase + l

                def body():
                    d = dv[l]
                    for v in range(VPR):
                        s = pl.ds(v * 16, 16)
                        plsc.addupdate(
                            acc.at[d, s],
                            jnp.maximum(xrows[r, s] + erows[r, s], 0.0))

                if guarded:
                    pl.when((r >= r_lo) & (r < r_hi))(body)
                else:
                    body()

            all_valid = (gbase >= r_lo) & (gbase + 16 <= r_hi)

            @pl.when(all_valid)
            def _():
                for l in range(16):
                    edge(l, False)

            @pl.when(jnp.logical_not(all_valid))
            def _():
                for l in range(16):
                    edge(l, True)

            return 0

        lax.fori_loop(0, CB // 16, group, 0)

    issue_idx(0, 0)
    wait_idx(0, 0)
    issue_data(0, 0)
    issue_idx(1, 1)

    npairs = (nch + 1) // 2

    def pair(j, _):
        for b in range(2):
            k = 2 * j + b
            o = 1 - b
            wait_idx(k + 1, o)
            issue_data(k + 1, o)
            wait_data(k, b)
            compute(k, b)
            issue_idx(k + 2, b)
        return 0

    lax.fori_loop(0, npairs, pair, 0)

    kk = 2 * npairs
    wait_data(kk, 0)
    wait_idx(kk + 1, 1)

    pltpu.sync_copy(acc, out_hbm.at[pl.ds(base_row, RPW)])


_sc_aggregate = functools.partial(
    pl.kernel,
    mesh=plsc.VectorSubcoreMesh(core_axis_name="c", subcore_axis_name="s"),
    out_type=jax.ShapeDtypeStruct((N_PAD, D), jnp.float32),
    scratch_types=[
        pltpu.VMEM((NW, 16), jnp.int32),
        pltpu.VMEM((CB,), jnp.int32),
        pltpu.VMEM((CB,), jnp.int32),
        pltpu.VMEM((CB,), jnp.int32),
        pltpu.VMEM((CB, D), jnp.float32),
        pltpu.VMEM((CB, D), jnp.float32),
        pltpu.VMEM((CB,), jnp.int32),
        pltpu.VMEM((CB,), jnp.int32),
        pltpu.VMEM((CB,), jnp.int32),
        pltpu.VMEM((CB, D), jnp.float32),
        pltpu.VMEM((CB, D), jnp.float32),
        pltpu.VMEM((RPW, D), jnp.float32),
        pltpu.SemaphoreType.DMA,
        pltpu.SemaphoreType.DMA,
        pltpu.SemaphoreType.DMA,
        pltpu.SemaphoreType.DMA,
    ],
)(_sc_body)


_BE = 2000


def _edge_mlp_body(ea_ref, we_ref, be_ref, out_ref):
    out_ref[...] = lax.dot_general(
        ea_ref[...], we_ref[...], (((1,), (1,)), ((), ())),
        preferred_element_type=jnp.float32) + be_ref[...]


def _edge_mlp(edge_attr, We, be):
    return pl.pallas_call(
        _edge_mlp_body,
        grid=(E // _BE,),
        in_specs=[
            pl.BlockSpec((_BE, ED), lambda i: (i, 0)),
            pl.BlockSpec((D, ED), lambda i: (0, 0)),
            pl.BlockSpec((1, D), lambda i: (0, 0)),
        ],
        out_specs=pl.BlockSpec((_BE, D), lambda i: (i, 0)),
        out_shape=jax.ShapeDtypeStruct((E, D), jnp.float32),
    )(edge_attr, We, be.reshape(1, D))


def _node_core(x, agg, wa, ba, wb, bb, g, bt, pa):
    h = x + agg
    h = jnp.maximum(
        lax.dot_general(h, wa, (((1,), (1,)), ((), ())),
                        preferred_element_type=jnp.float32) + ba, 0.0)
    h = lax.dot_general(h, wb, (((1,), (1,)), ((), ())),
                        preferred_element_type=jnp.float32) + bb
    mu = jnp.mean(h, axis=0, keepdims=True)
    var = jnp.mean((h - mu) * (h - mu), axis=0, keepdims=True)
    h = (h - mu) * lax.rsqrt(var + 1e-5) * g + bt
    return jnp.where(h > 0, h, pa * h)


def _node_mid_body(x_ref, a_ref, wa_ref, ba_ref, wb_ref, bb_ref,
                   g_ref, bt_ref, pa_ref, out_ref):
    out_ref[...] = _node_core(
        x_ref[...], a_ref[:N, :], wa_ref[...], ba_ref[...],
        wb_ref[...], bb_ref[...], g_ref[...], bt_ref[...], pa_ref[...])


def _node_last_body(x_ref, a_ref, wa_ref, ba_ref, wb_ref, bb_ref,
                    g_ref, bt_ref, pa_ref, wo_ref, bo_ref, out_ref):
    h = _node_core(
        x_ref[...], a_ref[:N, :], wa_ref[...], ba_ref[...],
        wb_ref[...], bb_ref[...], g_ref[...], bt_ref[...], pa_ref[...])
    out_ref[...] = lax.dot_general(
        h, wo_ref[...], (((1,), (1,)), ((), ())),
        preferred_element_type=jnp.float32) + bo_ref[...]


def _node_update(x, agg, wa, ba, wb, bb, g, bt, pa, wo=None, bo=None):
    row = lambda v: v.reshape(1, D)
    args = [x, agg, wa, row(ba), wb, row(bb), row(g), row(bt), row(pa)]
    body = _node_mid_body
    if wo is not None:
        args += [wo, row(bo)]
        body = _node_last_body
    return pl.pallas_call(
        body,
        out_shape=jax.ShapeDtypeStruct((N, D), jnp.float32),
    )(*args)


def kernel(x, edge_index, edge_attr,
           We1, be1, W1a, b1a, W1b, b1b, g1, bt1, pa1,
           We2, be2, W2a, b2a, W2b, b2b, g2, bt2, pa2,
           We3, be3, W3a, b3a, W3b, b3b, g3, bt3, pa3,
           Wo, bo):
    perm = jnp.argsort(edge_index[1]).astype(jnp.int32)
    src_s = jnp.take(edge_index[0], perm)
    dst_s = jnp.take(edge_index[1], perm)
    bins = jnp.arange(0, N_PAD + RPW, RPW, dtype=jnp.int32)
    b = jnp.searchsorted(dst_s, bins).astype(jnp.int32)
    bounds = jnp.zeros((NW, 16), jnp.int32).at[:, 0].set(b[:-1]).at[:, 1].set(b[1:])

    e1 = _edge_mlp(edge_attr, We1, be1)
    agg = _sc_aggregate(x, e1, src_s, dst_s, perm, bounds)
    h = _node_update(x, agg, W1a, b1a, W1b, b1b, g1, bt1, pa1)

    e2 = _edge_mlp(edge_attr, We2, be2)
    agg = _sc_aggregate(h, e2, src_s, dst_s, perm, bounds)
    h = _node_update(h, agg, W2a, b2a, W2b, b2b, g2, bt2, pa2)

    e3 = _edge_mlp(edge_attr, We3, be3)
    agg = _sc_aggregate(h, e3, src_s, dst_s, perm, bounds)
    return _node_update(h, agg, W3a, b3a, W3b, b3b, g3, bt3, pa3, Wo, bo)


# speedup vs baseline: 3.2272x; 1.7304x over previous
import functools

import jax
import jax.numpy as jnp
from jax import lax
from jax.experimental import pallas as pl
from jax.experimental.pallas import tpu as pltpu
from jax.experimental.pallas import tpu_sc as plsc

N = 10000
E = 320000
D = 128
ED = 16

NC = 2
NS = 16
NW = NC * NS
N_PAD = 10240
RPW = N_PAD // NW
RPT = 640
RPT_LAST = N - (NS - 1) * RPT
C = 120
CB = 128
VPR = D // 16


def _sc_body(x_hbm, e_hbm, src_hbm, dst_hbm, prm_hbm, bounds_hbm, out_hbm,
             bounds_v,
             src0, prm0, dst0, x0, e0,
             src1, prm1, dst1, x1, e1,
             acc, semi0, semi1, semd0, semd1):
    cid = lax.axis_index("c")
    sid = lax.axis_index("s")
    wid = sid * NC + cid
    base_row = wid * RPW

    srcb = (src0, src1)
    prmb = (prm0, prm1)
    dstb = (dst0, dst1)
    xb = (x0, x1)
    eb = (e0, e1)
    semi = (semi0, semi1)
    semd = (semd0, semd1)

    pltpu.sync_copy(bounds_hbm, bounds_v)
    bv = bounds_v[wid, pl.ds(0, 16)]
    b_lo = bv[0]
    b_hi = bv[1]

    zero = jnp.zeros((16,), jnp.float32)

    def zrow(r, _):
        for v in range(VPR):
            acc[r, pl.ds(v * 16, 16)] = zero
        return 0

    lax.fori_loop(0, RPW, zrow, 0)

    nch = (b_hi - b_lo + C - 1) // C

    def chunk_base(k):
        logical = b_lo + k * C
        return jnp.minimum((logical // 8) * 8, E - CB)

    def issue_idx(k, b):
        base = chunk_base(k)
        pltpu.async_copy(src_hbm.at[pl.ds(base, CB)], srcb[b], semi[b])
        pltpu.async_copy(prm_hbm.at[pl.ds(base, CB)], prmb[b], semi[b])
        pltpu.async_copy(dst_hbm.at[pl.ds(base, CB)], dstb[b], semi[b])

    def wait_idx(k, b):
        base = chunk_base(k)
        pltpu.make_async_copy(src_hbm.at[pl.ds(base, CB)], srcb[b], semi[b]).wait()
        pltpu.make_async_copy(prm_hbm.at[pl.ds(base, CB)], prmb[b], semi[b]).wait()
        pltpu.make_async_copy(dst_hbm.at[pl.ds(base, CB)], dstb[b], semi[b]).wait()

    def issue_data(k, b):
        pltpu.async_copy(x_hbm.at[srcb[b]], xb[b], semd[b])
        pltpu.async_copy(e_hbm.at[prmb[b]], eb[b], semd[b])

    def wait_data(k, b):
        pltpu.make_async_copy(x_hbm.at[srcb[b]], xb[b], semd[b]).wait()
        pltpu.make_async_copy(e_hbm.at[prmb[b]], eb[b], semd[b]).wait()

    def compute(k, b, carry):
        logical = b_lo + k * C
        base = chunk_base(k)
        r_lo = logical - base
        r_hi = jnp.minimum(logical + C, b_hi) - base
        xrows = xb[b]
        erows = eb[b]
        dst_idx = dstb[b]

        def group(g, carry):
            gbase = g * 16
            dv = dst_idx[pl.ds(gbase, 16)] - base_row
            d_prev, a = carry
            for l in range(16):
                r = gbase + l
                valid = (r >= r_lo) & (r < r_hi)
                d = dv[l]
                change = valid & (d != d_prev)
                d_store = jnp.where(valid, d, RPW)
                newa = []
                for v in range(VPR):
                    s = pl.ds(v * 16, 16)
                    val = jnp.maximum(xrows[r, s] + erows[r, s], 0.0)
                    newa.append(jnp.where(
                        change, val,
                        jnp.where(valid, a[v] + val, a[v])))
                a = tuple(newa)
                for v in range(VPR):
                    acc[d_store, pl.ds(v * 16, 16)] = a[v]
                d_prev = jnp.where(valid, d, d_prev)
            return d_prev, a

        return lax.fori_loop(0, CB // 16, lambda g, c: group(g, c), carry)

    issue_idx(0, 0)
    wait_idx(0, 0)
    issue_data(0, 0)
    issue_idx(1, 1)

    npairs = (nch + 1) // 2
    carry0 = (jnp.int32(-1), tuple(zero for _ in range(VPR)))

    def pair(j, carry):
        for b in range(2):
            k = 2 * j + b
            o = 1 - b
            wait_idx(k + 1, o)
            issue_data(k + 1, o)
            wait_data(k, b)
            carry = compute(k, b, carry)
            issue_idx(k + 2, b)
        return carry

    lax.fori_loop(0, npairs, pair, carry0)

    kk = 2 * npairs
    wait_data(kk, 0)
    wait_idx(kk + 1, 1)

    pltpu.sync_copy(acc.at[pl.ds(0, RPW)], out_hbm.at[pl.ds(base_row, RPW)])


_sc_aggregate = functools.partial(
    pl.kernel,
    mesh=plsc.VectorSubcoreMesh(core_axis_name="c", subcore_axis_name="s"),
    out_type=jax.ShapeDtypeStruct((N_PAD, D), jnp.float32),
    scratch_types=[
        pltpu.VMEM((NW, 16), jnp.int32),
        pltpu.VMEM((CB,), jnp.int32),
        pltpu.VMEM((CB,), jnp.int32),
        pltpu.VMEM((CB,), jnp.int32),
        pltpu.VMEM((CB, D), jnp.float32),
        pltpu.VMEM((CB, D), jnp.float32),
        pltpu.VMEM((CB,), jnp.int32),
        pltpu.VMEM((CB,), jnp.int32),
        pltpu.VMEM((CB,), jnp.int32),
        pltpu.VMEM((CB, D), jnp.float32),
        pltpu.VMEM((CB, D), jnp.float32),
        pltpu.VMEM((RPW + 8, D), jnp.float32),
        pltpu.SemaphoreType.DMA,
        pltpu.SemaphoreType.DMA,
        pltpu.SemaphoreType.DMA,
        pltpu.SemaphoreType.DMA,
    ],
)(_sc_body)


_BE = 2000


def _edge_mlp_body(ea_ref, we_ref, be_ref, out_ref):
    out_ref[...] = lax.dot_general(
        ea_ref[...], we_ref[...], (((1,), (1,)), ((), ())),
        preferred_element_type=jnp.float32) + be_ref[...]


def _edge_mlp(edge_attr, We, be):
    return pl.pallas_call(
        _edge_mlp_body,
        grid=(E // _BE,),
        in_specs=[
            pl.BlockSpec((_BE, ED), lambda i: (i, 0)),
            pl.BlockSpec((D, ED), lambda i: (0, 0)),
            pl.BlockSpec((1, D), lambda i: (0, 0)),
        ],
        out_specs=pl.BlockSpec((_BE, D), lambda i: (i, 0)),
        out_shape=jax.ShapeDtypeStruct((E, D), jnp.float32),
    )(edge_attr, We, be.reshape(1, D))


def _node_core(x, agg, wa, ba, wb, bb, g, bt, pa):
    h = x + agg
    h = jnp.maximum(
        lax.dot_general(h, wa, (((1,), (1,)), ((), ())),
                        preferred_element_type=jnp.float32) + ba, 0.0)
    h = lax.dot_general(h, wb, (((1,), (1,)), ((), ())),
                        preferred_element_type=jnp.float32) + bb
    mu = jnp.mean(h, axis=0, keepdims=True)
    var = jnp.mean((h - mu) * (h - mu), axis=0, keepdims=True)
    h = (h - mu) * lax.rsqrt(var + 1e-5) * g + bt
    return jnp.where(h > 0, h, pa * h)


def _node_mid_body(x_ref, a_ref, wa_ref, ba_ref, wb_ref, bb_ref,
                   g_ref, bt_ref, pa_ref, out_ref):
    out_ref[...] = _node_core(
        x_ref[...], a_ref[:N, :], wa_ref[...], ba_ref[...],
        wb_ref[...], bb_ref[...], g_ref[...], bt_ref[...], pa_ref[...])


def _node_last_body(x_ref, a_ref, wa_ref, ba_ref, wb_ref, bb_ref,
                    g_ref, bt_ref, pa_ref, wo_ref, bo_ref, out_ref):
    h = _node_core(
        x_ref[...], a_ref[:N, :], wa_ref[...], ba_ref[...],
        wb_ref[...], bb_ref[...], g_ref[...], bt_ref[...], pa_ref[...])
    out_ref[...] = lax.dot_general(
        h, wo_ref[...], (((1,), (1,)), ((), ())),
        preferred_element_type=jnp.float32) + bo_ref[...]


def _node_update(x, agg, wa, ba, wb, bb, g, bt, pa, wo=None, bo=None):
    row = lambda v: v.reshape(1, D)
    args = [x, agg, wa, row(ba), wb, row(bb), row(g), row(bt), row(pa)]
    body = _node_mid_body
    if wo is not None:
        args += [wo, row(bo)]
        body = _node_last_body
    return pl.pallas_call(
        body,
        out_shape=jax.ShapeDtypeStruct((N, D), jnp.float32),
    )(*args)


def kernel(x, edge_index, edge_attr,
           We1, be1, W1a, b1a, W1b, b1b, g1, bt1, pa1,
           We2, be2, W2a, b2a, W2b, b2b, g2, bt2, pa2,
           We3, be3, W3a, b3a, W3b, b3b, g3, bt3, pa3,
           Wo, bo):
    perm = jnp.argsort(edge_index[1]).astype(jnp.int32)
    src_s = jnp.take(edge_index[0], perm)
    dst_s = jnp.take(edge_index[1], perm)
    bins = jnp.arange(0, N_PAD + RPW, RPW, dtype=jnp.int32)
    b = jnp.searchsorted(dst_s, bins).astype(jnp.int32)
    bounds = jnp.zeros((NW, 16), jnp.int32).at[:, 0].set(b[:-1]).at[:, 1].set(b[1:])

    e1 = _edge_mlp(edge_attr, We1, be1)
    agg = _sc_aggregate(x, e1, src_s, dst_s, perm, bounds)
    h = _node_update(x, agg, W1a, b1a, W1b, b1b, g1, bt1, pa1)

    e2 = _edge_mlp(edge_attr, We2, be2)
    agg = _sc_aggregate(h, e2, src_s, dst_s, perm, bounds)
    h = _node_update(h, agg, W2a, b2a, W2b, b2b, g2, bt2, pa2)

    e3 = _edge_mlp(edge_attr, We3, be3)
    agg = _sc_aggregate(h, e3, src_s, dst_s, perm, bounds)
    return _node_update(h, agg, W3a, b3a, W3b, b3b, g3, bt3, pa3, Wo, bo)


# speedup vs baseline: 3.3785x; 1.0469x over previous
import functools

import jax
import jax.numpy as jnp
from jax import lax
from jax.experimental import pallas as pl
from jax.experimental.pallas import tpu as pltpu
from jax.experimental.pallas import tpu_sc as plsc

N = 10000
E = 320000
D = 128
ED = 16

NC = 2
NS = 16
NW = NC * NS
N_PAD = 10240
RPW = N_PAD // NW
RPT = 640
RPT_LAST = N - (NS - 1) * RPT
C = 120
CB = 128
VPR = D // 16


def _sc_body(x_hbm, e_hbm, src_hbm, dst_hbm, prm_hbm, bounds_hbm, out_hbm,
             bounds_v,
             src0, prm0, dst0, x0, e0,
             src1, prm1, dst1, x1, e1,
             acc, semi0, semi1, semd0, semd1):
    cid = lax.axis_index("c")
    sid = lax.axis_index("s")
    wid = sid * NC + cid
    base_row = wid * RPW

    srcb = (src0, src1)
    prmb = (prm0, prm1)
    dstb = (dst0, dst1)
    xb = (x0, x1)
    eb = (e0, e1)
    semi = (semi0, semi1)
    semd = (semd0, semd1)

    pltpu.sync_copy(bounds_hbm, bounds_v)
    bv = bounds_v[wid, pl.ds(0, 16)]
    b_lo = bv[0]
    b_hi = bv[1]

    zero = jnp.zeros((16,), jnp.float32)

    def zrow(r, _):
        for v in range(VPR):
            acc[r, pl.ds(v * 16, 16)] = zero
        return 0

    lax.fori_loop(0, RPW, zrow, 0)

    nch = (b_hi - b_lo + C - 1) // C

    def chunk_base(k):
        logical = b_lo + k * C
        return jnp.minimum((logical // 8) * 8, E - CB)

    def issue_idx(k, b):
        base = chunk_base(k)
        pltpu.async_copy(src_hbm.at[pl.ds(base, CB)], srcb[b], semi[b])
        pltpu.async_copy(prm_hbm.at[pl.ds(base, CB)], prmb[b], semi[b])
        pltpu.async_copy(dst_hbm.at[pl.ds(base, CB)], dstb[b], semi[b])

    def wait_idx(k, b):
        base = chunk_base(k)
        pltpu.make_async_copy(src_hbm.at[pl.ds(base, CB)], srcb[b], semi[b]).wait()
        pltpu.make_async_copy(prm_hbm.at[pl.ds(base, CB)], prmb[b], semi[b]).wait()
        pltpu.make_async_copy(dst_hbm.at[pl.ds(base, CB)], dstb[b], semi[b]).wait()

    def issue_data(k, b):
        pltpu.async_copy(x_hbm.at[srcb[b]], xb[b], semd[b])
        pltpu.async_copy(e_hbm.at[prmb[b]], eb[b], semd[b])

    def wait_data(k, b):
        pltpu.make_async_copy(x_hbm.at[srcb[b]], xb[b], semd[b]).wait()
        pltpu.make_async_copy(e_hbm.at[prmb[b]], eb[b], semd[b]).wait()

    def compute(k, b, carry):
        logical = b_lo + k * C
        base = chunk_base(k)
        r_lo = logical - base
        r_hi = jnp.minimum(logical + C, b_hi) - base
        xrows = xb[b]
        erows = eb[b]
        dst_idx = dstb[b]

        def group(g, carry):
            gbase = g * 16
            dv = dst_idx[pl.ds(gbase, 16)] - base_row
            d_prev, a = carry
            for l in range(16):
                r = gbase + l
                valid = (r >= r_lo) & (r < r_hi)
                d = dv[l]
                change = valid & (d != d_prev)
                d_store = jnp.where(valid, d, RPW)
                newa = []
                for v in range(VPR):
                    s = pl.ds(v * 16, 16)
                    val = jnp.maximum(xrows[r, s] + erows[r, s], 0.0)
                    newa.append(jnp.where(
                        change, val,
                        jnp.where(valid, a[v] + val, a[v])))
                a = tuple(newa)
                for v in range(VPR):
                    acc[d_store, pl.ds(v * 16, 16)] = a[v]
                d_prev = jnp.where(valid, d, d_prev)
            return d_prev, a

        return lax.fori_loop(0, CB // 16, lambda g, c: group(g, c), carry)

    issue_idx(0, 0)
    wait_idx(0, 0)
    issue_data(0, 0)
    issue_idx(1, 1)

    npairs = (nch + 1) // 2
    carry0 = (jnp.int32(-1), tuple(zero for _ in range(VPR)))

    def pair(j, carry):
        for b in range(2):
            k = 2 * j + b
            o = 1 - b
            wait_idx(k + 1, o)
            issue_data(k + 1, o)
            wait_data(k, b)
            carry = compute(k, b, carry)
            issue_idx(k + 2, b)
        return carry

    lax.fori_loop(0, npairs, pair, carry0)

    kk = 2 * npairs
    wait_data(kk, 0)
    wait_idx(kk + 1, 1)

    pltpu.sync_copy(acc.at[pl.ds(0, RPW)], out_hbm.at[pl.ds(base_row, RPW)])


_sc_aggregate = functools.partial(
    pl.kernel,
    mesh=plsc.VectorSubcoreMesh(core_axis_name="c", subcore_axis_name="s"),
    out_type=jax.ShapeDtypeStruct((N_PAD, D), jnp.float32),
    scratch_types=[
        pltpu.VMEM((NW, 16), jnp.int32),
        pltpu.VMEM((CB,), jnp.int32),
        pltpu.VMEM((CB,), jnp.int32),
        pltpu.VMEM((CB,), jnp.int32),
        pltpu.VMEM((CB, D), jnp.float32),
        pltpu.VMEM((CB, D), jnp.float32),
        pltpu.VMEM((CB,), jnp.int32),
        pltpu.VMEM((CB,), jnp.int32),
        pltpu.VMEM((CB,), jnp.int32),
        pltpu.VMEM((CB, D), jnp.float32),
        pltpu.VMEM((CB, D), jnp.float32),
        pltpu.VMEM((RPW + 8, D), jnp.float32),
        pltpu.SemaphoreType.DMA,
        pltpu.SemaphoreType.DMA,
        pltpu.SemaphoreType.DMA,
        pltpu.SemaphoreType.DMA,
    ],
)(_sc_body)


_BE = 2000


def _edge_mlp_body(ea_ref, we_ref, be_ref, out_ref):
    out_ref[...] = lax.dot_general(
        ea_ref[...], we_ref[...], (((1,), (1,)), ((), ())),
        preferred_element_type=jnp.float32) + be_ref[...]


def _edge_mlp(edge_attr, We, be):
    return pl.pallas_call(
        _edge_mlp_body,
        grid=(E // _BE,),
        in_specs=[
            pl.BlockSpec((_BE, ED), lambda i: (i, 0)),
            pl.BlockSpec((D, ED), lambda i: (0, 0)),
            pl.BlockSpec((1, D), lambda i: (0, 0)),
        ],
        out_specs=pl.BlockSpec((_BE, D), lambda i: (i, 0)),
        out_shape=jax.ShapeDtypeStruct((E, D), jnp.float32),
    )(edge_attr, We, be.reshape(1, D))


def _node_core(x, agg, wa, ba, wb, bb, g, bt, pa):
    h = x + agg
    h = jnp.maximum(
        lax.dot_general(h, wa, (((1,), (1,)), ((), ())),
                        preferred_element_type=jnp.float32) + ba, 0.0)
    h = lax.dot_general(h, wb, (((1,), (1,)), ((), ())),
                        preferred_element_type=jnp.float32) + bb
    mu = jnp.mean(h, axis=0, keepdims=True)
    var = jnp.mean((h - mu) * (h - mu), axis=0, keepdims=True)
    h = (h - mu) * lax.rsqrt(var + 1e-5) * g + bt
    return jnp.where(h > 0, h, pa * h)


def _node_mid_body(x_ref, a_ref, wa_ref, ba_ref, wb_ref, bb_ref,
                   g_ref, bt_ref, pa_ref, out_ref):
    out_ref[...] = _node_core(
        x_ref[...], a_ref[:N, :], wa_ref[...], ba_ref[...],
        wb_ref[...], bb_ref[...], g_ref[...], bt_ref[...], pa_ref[...])


def _node_last_body(x_ref, a_ref, wa_ref, ba_ref, wb_ref, bb_ref,
                    g_ref, bt_ref, pa_ref, wo_ref, bo_ref, out_ref):
    h = _node_core(
        x_ref[...], a_ref[:N, :], wa_ref[...], ba_ref[...],
        wb_ref[...], bb_ref[...], g_ref[...], bt_ref[...], pa_ref[...])
    out_ref[...] = lax.dot_general(
        h, wo_ref[...], (((1,), (1,)), ((), ())),
        preferred_element_type=jnp.float32) + bo_ref[...]


def _node_update(x, agg, wa, ba, wb, bb, g, bt, pa, wo=None, bo=None):
    row = lambda v: v.reshape(1, D)
    args = [x, agg, wa, row(ba), wb, row(bb), row(g), row(bt), row(pa)]
    body = _node_mid_body
    if wo is not None:
        args += [wo, row(bo)]
        body = _node_last_body
    return pl.pallas_call(
        body,
        out_shape=jax.ShapeDtypeStruct((N, D), jnp.float32),
    )(*args)


def kernel(x, edge_index, edge_attr,
           We1, be1, W1a, b1a, W1b, b1b, g1, bt1, pa1,
           We2, be2, W2a, b2a, W2b, b2b, g2, bt2, pa2,
           We3, be3, W3a, b3a, W3b, b3b, g3, bt3, pa3,
           Wo, bo):
    dst_s, perm = lax.sort_key_val(
        edge_index[1], jnp.arange(E, dtype=jnp.int32), is_stable=False)
    src_s = jnp.take(edge_index[0], perm)
    bins = jnp.arange(0, N_PAD + RPW, RPW, dtype=jnp.int32)
    b = jnp.searchsorted(dst_s, bins).astype(jnp.int32)
    bounds = jnp.zeros((NW, 16), jnp.int32).at[:, 0].set(b[:-1]).at[:, 1].set(b[1:])

    e1 = _edge_mlp(edge_attr, We1, be1)
    agg = _sc_aggregate(x, e1, src_s, dst_s, perm, bounds)
    h = _node_update(x, agg, W1a, b1a, W1b, b1b, g1, bt1, pa1)

    e2 = _edge_mlp(edge_attr, We2, be2)
    agg = _sc_aggregate(h, e2, src_s, dst_s, perm, bounds)
    h = _node_update(h, agg, W2a, b2a, W2b, b2b, g2, bt2, pa2)

    e3 = _edge_mlp(edge_attr, We3, be3)
    agg = _sc_aggregate(h, e3, src_s, dst_s, perm, bounds)
    return _node_update(h, agg, W3a, b3a, W3b, b3b, g3, bt3, pa3, Wo, bo)
